# baseline (speedup 1.0000x reference)
import functools

import jax
import jax.numpy as jnp
import numpy as np
from jax.experimental import pallas as pl
from jax.experimental.pallas import tpu as pltpu

N = 10000
NP = 10240
E = 160000
IN_DIM = 256
H = 128
NH = 4
DH = 16
MAXC = 0.15

BM = 1000
NBLK = N // BM
BMA = 400
NQB = N // BMA


def _front_body(x_ref, wp_ref, bp_ref, w1_ref, hw_ref):
    h = jnp.maximum(
        jnp.dot(x_ref[...], wp_ref[...], preferred_element_type=jnp.float32)
        + bp_ref[...], 0.0)
    hw_ref[...] = jnp.dot(h, w1_ref[...], preferred_element_type=jnp.float32)


def _front(x, Wp, bp, W1):
    return pl.pallas_call(
        _front_body,
        grid=(NBLK,),
        in_specs=[
            pl.BlockSpec((BM, IN_DIM), lambda i: (i, 0)),
            pl.BlockSpec((IN_DIM, H), lambda i: (0, 0)),
            pl.BlockSpec((1, H), lambda i: (0, 0)),
            pl.BlockSpec((H, H), lambda i: (0, 0)),
        ],
        out_specs=pl.BlockSpec((BM, H), lambda i: (i, 0)),
        out_shape=jax.ShapeDtypeStruct((N, H), jnp.float32),
    )(x, Wp, bp.reshape(1, H), W1)


def _mm_body(x_ref, w_ref, b_ref, o_ref):
    o_ref[...] = jnp.dot(x_ref[...], w_ref[...],
                         preferred_element_type=jnp.float32) + b_ref[...]


def _matmul(x, W, b):
    din, dout = W.shape
    return pl.pallas_call(
        _mm_body,
        grid=(NBLK,),
        in_specs=[
            pl.BlockSpec((BM, din), lambda i: (i, 0)),
            pl.BlockSpec((din, dout), lambda i: (0, 0)),
            pl.BlockSpec((1, dout), lambda i: (0, 0)),
        ],
        out_specs=pl.BlockSpec((BM, dout), lambda i: (i, 0)),
        out_shape=jax.ShapeDtypeStruct((N, dout), jnp.float32),
    )(x, W, b.reshape(1, dout))


def _attn_body(q_ref, kv_ref, wo_ref, ob_ref, w1_ref, b1_ref, w2_ref, b2_ref,
               y_ref, s_buf):
    d = H // 2
    parts = []
    for h in range(NH):
        q = q_ref[:, h * DH:(h + 1) * DH] * 0.25

        def p1(j, m, h=h, q=q):
            k = kv_ref[pl.ds(j * 512, 512), d + h * DH:d + (h + 1) * DH]
            st = jax.lax.dot_general(k, q, (((1,), (1,)), ((), ())),
                                     preferred_element_type=jnp.float32)
            s_buf[pl.ds(j * 512, 512), :] = st
            return jnp.maximum(m, jnp.max(st, axis=0, keepdims=True))

        m = jax.lax.fori_loop(
            0, 19, p1, jnp.full((1, BMA), -jnp.inf, jnp.float32))
        kt = kv_ref[pl.ds(9728, 272), d + h * DH:d + (h + 1) * DH]
        st = jax.lax.dot_general(kt, q, (((1,), (1,)), ((), ())),
                                 preferred_element_type=jnp.float32)
        s_buf[pl.ds(9728, 272), :] = st
        m = jnp.maximum(m, jnp.max(st, axis=0, keepdims=True))
        s_buf[pl.ds(10000, 240), :] = jnp.zeros((240, BMA), jnp.float32)

        def p2(b, _, m=m):
            rows = pl.ds(b * 128, 128)
            s_buf[rows, :] = jnp.exp(s_buf[rows, :] - m)
            return 0

        jax.lax.fori_loop(0, 78, p2, 0)
        rows = pl.ds(9984, 16)
        s_buf[rows, :] = jnp.exp(s_buf[rows, :] - m)

        acc = [jnp.zeros((128, BMA), jnp.float32) for _ in range(2)]
        for t in range(5):
            ch = jnp.zeros((128, BMA), jnp.float32)
            for u in range(16):
                ch = ch + s_buf[pl.ds((16 * t + u) * 128, 128), :]
            acc[t % 2] = acc[t % 2] + ch
        a = acc[0] + acc[1]
        for dd in (64, 32, 16, 8, 4, 2, 1):
            a = a[:dd, :] + a[dd:2 * dd, :]
        l = a

        def p3a(j, _, l=l):
            rows = pl.ds(j * 512, 512)
            s_buf[rows, :] = s_buf[rows, :] / l
            return 0

        jax.lax.fori_loop(0, 20, p3a, 0)
        v_all = kv_ref[:, 2 * d + h * DH:2 * d + (h + 1) * DH]
        o = jax.lax.dot_general(s_buf[...], v_all, (((0,), (0,)), ((), ())),
                                preferred_element_type=jnp.float32)
        parts.append(o)

    att = jnp.concatenate(parts, axis=1)
    o2 = jnp.dot(att, wo_ref[...], preferred_element_type=jnp.float32) \
        + ob_ref[...]
    t = jnp.maximum(
        jnp.dot(o2, w1_ref[...], preferred_element_type=jnp.float32)
        + b1_ref[...], 0.0)
    y_ref[...] = jnp.dot(t, w2_ref[...], preferred_element_type=jnp.float32) \
        + b2_ref[0, 0]


def _attention(qkv, qkv_pad, WoT, ob, Wh1, bh1, Wh2, bh2):
    d = H // 2
    return pl.pallas_call(
        _attn_body,
        grid=(NQB,),
        in_specs=[
            pl.BlockSpec((BMA, 3 * d), lambda i: (i, 0)),
            pl.BlockSpec((NP, 3 * d), lambda i: (0, 0)),
            pl.BlockSpec((d, d), lambda i: (0, 0)),
            pl.BlockSpec((1, d), lambda i: (0, 0)),
            pl.BlockSpec((d, H // 4), lambda i: (0, 0)),
            pl.BlockSpec((1, H // 4), lambda i: (0, 0)),
            pl.BlockSpec((H // 4, 1), lambda i: (0, 0)),
            pl.BlockSpec((1, 1), lambda i: (0, 0)),
        ],
        out_specs=pl.BlockSpec((BMA, 1), lambda i: (i, 0)),
        out_shape=jax.ShapeDtypeStruct((N, 1), jnp.float32),
        scratch_shapes=[pltpu.VMEM((NP, BMA), jnp.float32)],
    )(qkv, qkv_pad, WoT, ob.reshape(1, d), Wh1, bh1.reshape(1, H // 4), Wh2,
      bh2.reshape(1, 1))


def _gcn_agg(hw, b, src, dst, dinv, norm_e):
    s = jnp.concatenate([src, jnp.arange(N, dtype=src.dtype)])
    dd = jnp.concatenate([dst, jnp.arange(N, dtype=src.dtype)])
    out = jnp.zeros_like(hw).at[dd].add(hw[s] * norm_e)
    return out + b


def _bn_relu(x, g, be):
    m = jnp.mean(x, axis=0)
    v = jnp.mean((x - m) ** 2, axis=0)
    return jnp.maximum((x - m) / jnp.sqrt(v + 1e-5) * g + be, 0.0)


def kernel(x, edge_index, idm_baseline, Wp, bp, W1, b1, W2, b2, W3, b3, g1,
           be1, g2, be2, in_w, in_b, out_w, out_b, Wh1, bh1, Wh2, bh2):
    src, dst = edge_index[0], edge_index[1]
    loop = jnp.arange(N, dtype=src.dtype)
    s_all = jnp.concatenate([src, loop])
    d_all = jnp.concatenate([dst, loop])
    deg = jnp.zeros((N,), jnp.float32).at[d_all].add(1.0)
    dinv = jax.lax.rsqrt(jnp.maximum(deg, 1.0))
    norm_e = (dinv[s_all] * dinv[d_all])[:, None]

    hw1 = _front(x, Wp, bp, W1)
    h1 = _bn_relu(_gcn_agg(hw1, b1, src, dst, dinv, norm_e), g1, be1)

    hw2 = _matmul(h1, W2, b2 * 0)
    h2 = _bn_relu(_gcn_agg(hw2, b2, src, dst, dinv, norm_e), g2, be2) + h1

    hw3 = _matmul(h2, W3, b3 * 0)
    h3 = jnp.maximum(_gcn_agg(hw3, b3, src, dst, dinv, norm_e), 0.0)

    qkv = _matmul(h3, in_w.T, in_b)
    qkv_pad = jnp.pad(qkv, ((0, NP - N), (0, 0)))

    y = _attention(qkv, qkv_pad, out_w.T, out_b, Wh1, bh1, Wh2, bh2)

    c = jnp.tanh(y) * MAXC
    c = c - jnp.mean(c)
    base = idm_baseline[:, 0]
    fin = jnp.clip(base + c[:, 0], 0.0, 1.0)
    c = (fin - base)[:, None]
    return c - jnp.mean(c)
